# Initial kernel scaffold; baseline (speedup 1.0000x reference)
import jax
import jax.numpy as jnp
from jax.experimental import pallas as pl


def kernel(x, edge_index, batch, W1_0, b1_0, W2_0, b2_0, W1_1, b1_1, W2_1, b2_1, W_out, b_out):
    raise NotImplementedError("write your pallas kernel here")


# speedup vs baseline: 10.2891x; 10.2891x over previous
import functools

import jax
import jax.numpy as jnp
from jax import lax
from jax.experimental import pallas as pl
from jax.experimental.pallas import tpu as pltpu
from jax.experimental.pallas import tpu_sc as plsc

N_NODES = 10000
N_EDGES = 320000
D_FEAT = 128
HID = 16
NG = 64

NODES_P = 10240
EDGES_P = 327680
NCORE = 2
NSUB = 16
NW = NCORE * NSUB
CHUNK = 128
NCHUNK = EDGES_P // (NW * CHUNK)
RPT = NODES_P // NSUB
RB = 1280
GRID = NODES_P // RB


def _seg_sum16(y_hbm, src2d, dst2d):
    mesh = plsc.VectorSubcoreMesh(core_axis_name="c", subcore_axis_name="s")

    @functools.partial(
        pl.kernel,
        out_type=jax.ShapeDtypeStruct((NCORE, NODES_P, HID), jnp.float32),
        mesh=mesh,
        compiler_params=pltpu.CompilerParams(use_tc_tiling_on_sc=False),
        scratch_types=[
            pltpu.VMEM((NCHUNK, CHUNK), jnp.int32),
            pltpu.VMEM((NCHUNK, CHUNK), jnp.int32),
            pltpu.VMEM((CHUNK, HID), jnp.float32),
            pltpu.VMEM((RPT, HID), jnp.float32),
            pltpu.VMEM_SHARED((NODES_P, HID), jnp.float32),
            pltpu.SemaphoreType.DMA,
        ],
    )
    def k(y_ref, src_ref, dst_ref, out_ref, src_v, dst_v, rows_v, stage_v,
          acc_sh, sem):
        cid = lax.axis_index("c")
        sid = lax.axis_index("s")
        wid = cid * NSUB + sid

        def zrow(i, carry):
            stage_v[i, :] = jnp.zeros((HID,), jnp.float32)
            return carry
        lax.fori_loop(0, RPT, zrow, 0)
        pltpu.sync_copy(stage_v, acc_sh.at[pl.ds(sid * RPT, RPT)])

        pltpu.sync_copy(src_ref.at[pl.ds(wid * NCHUNK, NCHUNK)], src_v)
        pltpu.sync_copy(dst_ref.at[pl.ds(wid * NCHUNK, NCHUNK)], dst_v)

        plsc.subcore_barrier()

        def chunk(j, carry):
            pltpu.async_copy(y_ref.at[src_v.at[j]], rows_v, sem).wait()
            pltpu.sync_copy(rows_v, acc_sh.at[dst_v.at[j]], add=True)
            return carry
        lax.fori_loop(0, NCHUNK, chunk, 0)

        plsc.subcore_barrier()

        pltpu.sync_copy(acc_sh.at[pl.ds(sid * RPT, RPT)], stage_v)
        pltpu.sync_copy(stage_v, out_ref.at[cid, pl.ds(sid * RPT, RPT)])

    return k(y_hbm, src2d, dst2d)


def _proj0(x_pad, w1):
    def body(x_ref, w_ref, o_ref):
        o_ref[...] = jnp.dot(x_ref[...], w_ref[...],
                             preferred_element_type=jnp.float32)

    return pl.pallas_call(
        body,
        grid=(GRID,),
        in_specs=[
            pl.BlockSpec((RB, D_FEAT), lambda i: (i, 0)),
            pl.BlockSpec((D_FEAT, HID), lambda i: (0, 0)),
        ],
        out_specs=pl.BlockSpec((RB, HID), lambda i: (i, 0)),
        out_shape=jax.ShapeDtypeStruct((NODES_P, HID), jnp.float32),
    )(x_pad, w1)


def _mid(segp0, y0, b1_0, w2_0, b2_0, w1_1):
    def body(sp_ref, y_ref, b1_ref, w2_ref, b2_ref, w11_ref, o_ref):
        seg = sp_ref[0] + sp_ref[1]
        t = jnp.maximum(seg + y_ref[...] + b1_ref[...], 0.0)
        t = jnp.dot(t, w2_ref[...], preferred_element_type=jnp.float32)
        t = jnp.maximum(t + b2_ref[...], 0.0)
        o_ref[...] = jnp.dot(t, w11_ref[...],
                             preferred_element_type=jnp.float32)

    return pl.pallas_call(
        body,
        grid=(GRID,),
        in_specs=[
            pl.BlockSpec((NCORE, RB, HID), lambda i: (0, i, 0)),
            pl.BlockSpec((RB, HID), lambda i: (i, 0)),
            pl.BlockSpec((1, HID), lambda i: (0, 0)),
            pl.BlockSpec((HID, HID), lambda i: (0, 0)),
            pl.BlockSpec((1, HID), lambda i: (0, 0)),
            pl.BlockSpec((HID, HID), lambda i: (0, 0)),
        ],
        out_specs=pl.BlockSpec((RB, HID), lambda i: (i, 0)),
        out_shape=jax.ShapeDtypeStruct((NODES_P, HID), jnp.float32),
    )(segp0, y0, b1_0, w2_0, b2_0, w1_1)


def _head(segp1, y1, b1_1, w2_1, b2_1, batch_r, w_out, b_out):
    def body(sp_ref, y_ref, b1_ref, w2_ref, b2_ref, bt_ref, wo_ref, bo_ref,
             o_ref, sums, cnts):
        i = pl.program_id(0)

        @pl.when(i == 0)
        def _init():
            sums[...] = jnp.zeros_like(sums)
            cnts[...] = jnp.zeros_like(cnts)

        seg = sp_ref[0] + sp_ref[1]
        t = jnp.maximum(seg + y_ref[...] + b1_ref[...], 0.0)
        h2 = jnp.dot(t, w2_ref[...], preferred_element_type=jnp.float32)
        h2 = h2 + b2_ref[...]

        bvals = bt_ref[0]
        gids = lax.broadcasted_iota(jnp.int32, (NG, RB), 0)
        onehot = (bvals == gids).astype(jnp.float32)
        sums[...] += jnp.dot(onehot, h2, preferred_element_type=jnp.float32)
        cnts[...] += jnp.dot(onehot, jnp.ones((RB, HID), jnp.float32),
                             preferred_element_type=jnp.float32)

        @pl.when(i == GRID - 1)
        def _final():
            pooled = sums[...] / jnp.maximum(cnts[...], 1.0)
            g = jnp.maximum(pooled, 0.0)
            o_ref[...] = jnp.dot(g, wo_ref[...],
                                 preferred_element_type=jnp.float32)
            o_ref[...] += bo_ref[...]

    return pl.pallas_call(
        body,
        grid=(GRID,),
        in_specs=[
            pl.BlockSpec((NCORE, RB, HID), lambda i: (0, i, 0)),
            pl.BlockSpec((RB, HID), lambda i: (i, 0)),
            pl.BlockSpec((1, HID), lambda i: (0, 0)),
            pl.BlockSpec((HID, HID), lambda i: (0, 0)),
            pl.BlockSpec((1, HID), lambda i: (0, 0)),
            pl.BlockSpec((1, 1, RB), lambda i: (i, 0, 0)),
            pl.BlockSpec((HID, 1), lambda i: (0, 0)),
            pl.BlockSpec((1, 1), lambda i: (0, 0)),
        ],
        out_specs=pl.BlockSpec((NG, 1), lambda i: (0, 0)),
        out_shape=jax.ShapeDtypeStruct((NG, 1), jnp.float32),
        scratch_shapes=[
            pltpu.VMEM((NG, HID), jnp.float32),
            pltpu.VMEM((NG, HID), jnp.float32),
        ],
    )(segp1, y1, b1_1, w2_1, b2_1, batch_r, w_out, b_out)


def kernel(x, edge_index, batch, W1_0, b1_0, W2_0, b2_0, W1_1, b1_1, W2_1,
           b2_1, W_out, b_out):
    x_pad = jnp.pad(x, ((0, NODES_P - N_NODES), (0, 0)))
    src = edge_index[0]
    dst = edge_index[1]
    pad_e = EDGES_P - N_EDGES
    src2d = jnp.concatenate(
        [src, jnp.zeros((pad_e,), jnp.int32)]).reshape(EDGES_P // CHUNK, CHUNK)
    dst2d = jnp.concatenate(
        [dst, jnp.full((pad_e,), N_NODES, jnp.int32)]).reshape(
            EDGES_P // CHUNK, CHUNK)
    batch_r = jnp.concatenate(
        [batch, jnp.full((NODES_P - N_NODES,), NG, jnp.int32)]).reshape(
            GRID, 1, RB)

    y0 = _proj0(x_pad, W1_0)
    segp0 = _seg_sum16(y0, src2d, dst2d)
    y1 = _mid(segp0, y0, b1_0.reshape(1, HID), W2_0, b2_0.reshape(1, HID),
              W1_1)
    segp1 = _seg_sum16(y1, src2d, dst2d)
    out = _head(segp1, y1, b1_1.reshape(1, HID), W2_1, b2_1.reshape(1, HID),
                batch_r, W_out, b_out.reshape(1, 1))
    return jnp.reshape(out, (-1,))


# speedup vs baseline: 12.8861x; 1.2524x over previous
import functools

import jax
import jax.numpy as jnp
from jax import lax
from jax.experimental import pallas as pl
from jax.experimental.pallas import tpu as pltpu
from jax.experimental.pallas import tpu_sc as plsc

N_NODES = 10000
N_EDGES = 320000
D_FEAT = 128
HID = 16
NG = 64

NODES_P = 10240
EDGES_P = 327680
NCORE = 2
NSUB = 16
NW = NCORE * NSUB
CHUNK = 128
NCHUNK = EDGES_P // (NW * CHUNK)
RPT = NODES_P // NSUB
KBUF = 8
RB = 1280
GRID = NODES_P // RB


def _seg_sum16(y_hbm, src2d, dst2d):
    mesh = plsc.VectorSubcoreMesh(core_axis_name="c", subcore_axis_name="s")

    @functools.partial(
        pl.kernel,
        out_type=jax.ShapeDtypeStruct((NCORE, NODES_P, HID), jnp.float32),
        mesh=mesh,
        compiler_params=pltpu.CompilerParams(use_tc_tiling_on_sc=False),
        scratch_types=[
            pltpu.VMEM((NCHUNK, CHUNK), jnp.int32),
            pltpu.VMEM((NCHUNK, CHUNK), jnp.int32),
            pltpu.VMEM((KBUF, CHUNK, HID), jnp.float32),
            pltpu.VMEM((RPT, HID), jnp.float32),
            pltpu.VMEM_SHARED((NODES_P, HID), jnp.float32),
            pltpu.SemaphoreType.DMA,
            pltpu.SemaphoreType.DMA,
        ],
    )
    def k(y_ref, src_ref, dst_ref, out_ref, src_v, dst_v, rows_v, stage_v,
          acc_sh, gsem, ssem):
        cid = lax.axis_index("c")
        sid = lax.axis_index("s")
        wid = cid * NSUB + sid

        def zrow(i, carry):
            stage_v[i, :] = jnp.zeros((HID,), jnp.float32)
            return carry
        lax.fori_loop(0, RPT, zrow, 0)
        pltpu.sync_copy(stage_v, acc_sh.at[pl.ds(sid * RPT, RPT)])

        pltpu.sync_copy(src_ref.at[pl.ds(wid * NCHUNK, NCHUNK)], src_v)
        pltpu.sync_copy(dst_ref.at[pl.ds(wid * NCHUNK, NCHUNK)], dst_v)

        plsc.subcore_barrier()

        def group(g, carry):
            j0 = g * KBUF
            descs = []
            for b in range(KBUF):
                descs.append(pltpu.async_copy(
                    y_ref.at[src_v.at[j0 + b]], rows_v.at[b], gsem))
            for b in range(KBUF):
                descs[b].wait()
            sdescs = []
            for b in range(KBUF):
                sdescs.append(pltpu.async_copy(
                    rows_v.at[b], acc_sh.at[dst_v.at[j0 + b]], ssem,
                    add=True))
            for b in range(KBUF):
                sdescs[b].wait()
            return carry
        lax.fori_loop(0, NCHUNK // KBUF, group, 0)

        plsc.subcore_barrier()

        pltpu.sync_copy(acc_sh.at[pl.ds(sid * RPT, RPT)], stage_v)
        pltpu.sync_copy(stage_v, out_ref.at[cid, pl.ds(sid * RPT, RPT)])

    return k(y_hbm, src2d, dst2d)


def _proj0(x_pad, w1):
    def body(x_ref, w_ref, o_ref):
        o_ref[...] = jnp.dot(x_ref[...], w_ref[...],
                             preferred_element_type=jnp.float32)

    return pl.pallas_call(
        body,
        grid=(GRID,),
        in_specs=[
            pl.BlockSpec((RB, D_FEAT), lambda i: (i, 0)),
            pl.BlockSpec((D_FEAT, HID), lambda i: (0, 0)),
        ],
        out_specs=pl.BlockSpec((RB, HID), lambda i: (i, 0)),
        out_shape=jax.ShapeDtypeStruct((NODES_P, HID), jnp.float32),
    )(x_pad, w1)


def _mid(segp0, y0, b1_0, w2_0, b2_0, w1_1):
    def body(sp_ref, y_ref, b1_ref, w2_ref, b2_ref, w11_ref, o_ref):
        seg = sp_ref[0] + sp_ref[1]
        t = jnp.maximum(seg + y_ref[...] + b1_ref[...], 0.0)
        t = jnp.dot(t, w2_ref[...], preferred_element_type=jnp.float32)
        t = jnp.maximum(t + b2_ref[...], 0.0)
        o_ref[...] = jnp.dot(t, w11_ref[...],
                             preferred_element_type=jnp.float32)

    return pl.pallas_call(
        body,
        grid=(GRID,),
        in_specs=[
            pl.BlockSpec((NCORE, RB, HID), lambda i: (0, i, 0)),
            pl.BlockSpec((RB, HID), lambda i: (i, 0)),
            pl.BlockSpec((1, HID), lambda i: (0, 0)),
            pl.BlockSpec((HID, HID), lambda i: (0, 0)),
            pl.BlockSpec((1, HID), lambda i: (0, 0)),
            pl.BlockSpec((HID, HID), lambda i: (0, 0)),
        ],
        out_specs=pl.BlockSpec((RB, HID), lambda i: (i, 0)),
        out_shape=jax.ShapeDtypeStruct((NODES_P, HID), jnp.float32),
    )(segp0, y0, b1_0, w2_0, b2_0, w1_1)


def _head(segp1, y1, b1_1, w2_1, b2_1, batch_r, w_out, b_out):
    def body(sp_ref, y_ref, b1_ref, w2_ref, b2_ref, bt_ref, wo_ref, bo_ref,
             o_ref, sums, cnts):
        i = pl.program_id(0)

        @pl.when(i == 0)
        def _init():
            sums[...] = jnp.zeros_like(sums)
            cnts[...] = jnp.zeros_like(cnts)

        seg = sp_ref[0] + sp_ref[1]
        t = jnp.maximum(seg + y_ref[...] + b1_ref[...], 0.0)
        h2 = jnp.dot(t, w2_ref[...], preferred_element_type=jnp.float32)
        h2 = h2 + b2_ref[...]

        bvals = bt_ref[0]
        gids = lax.broadcasted_iota(jnp.int32, (NG, RB), 0)
        onehot = (bvals == gids).astype(jnp.float32)
        sums[...] += jnp.dot(onehot, h2, preferred_element_type=jnp.float32)
        cnts[...] += jnp.dot(onehot, jnp.ones((RB, HID), jnp.float32),
                             preferred_element_type=jnp.float32)

        @pl.when(i == GRID - 1)
        def _final():
            pooled = sums[...] / jnp.maximum(cnts[...], 1.0)
            g = jnp.maximum(pooled, 0.0)
            o_ref[...] = jnp.dot(g, wo_ref[...],
                                 preferred_element_type=jnp.float32)
            o_ref[...] += bo_ref[...]

    return pl.pallas_call(
        body,
        grid=(GRID,),
        in_specs=[
            pl.BlockSpec((NCORE, RB, HID), lambda i: (0, i, 0)),
            pl.BlockSpec((RB, HID), lambda i: (i, 0)),
            pl.BlockSpec((1, HID), lambda i: (0, 0)),
            pl.BlockSpec((HID, HID), lambda i: (0, 0)),
            pl.BlockSpec((1, HID), lambda i: (0, 0)),
            pl.BlockSpec((1, 1, RB), lambda i: (i, 0, 0)),
            pl.BlockSpec((HID, 1), lambda i: (0, 0)),
            pl.BlockSpec((1, 1), lambda i: (0, 0)),
        ],
        out_specs=pl.BlockSpec((NG, 1), lambda i: (0, 0)),
        out_shape=jax.ShapeDtypeStruct((NG, 1), jnp.float32),
        scratch_shapes=[
            pltpu.VMEM((NG, HID), jnp.float32),
            pltpu.VMEM((NG, HID), jnp.float32),
        ],
    )(segp1, y1, b1_1, w2_1, b2_1, batch_r, w_out, b_out)


def kernel(x, edge_index, batch, W1_0, b1_0, W2_0, b2_0, W1_1, b1_1, W2_1,
           b2_1, W_out, b_out):
    x_pad = jnp.pad(x, ((0, NODES_P - N_NODES), (0, 0)))
    src = edge_index[0]
    dst = edge_index[1]
    pad_e = EDGES_P - N_EDGES
    src2d = jnp.concatenate(
        [src, jnp.zeros((pad_e,), jnp.int32)]).reshape(EDGES_P // CHUNK, CHUNK)
    dst2d = jnp.concatenate(
        [dst, jnp.full((pad_e,), N_NODES, jnp.int32)]).reshape(
            EDGES_P // CHUNK, CHUNK)
    batch_r = jnp.concatenate(
        [batch, jnp.full((NODES_P - N_NODES,), NG, jnp.int32)]).reshape(
            GRID, 1, RB)

    y0 = _proj0(x_pad, W1_0)
    segp0 = _seg_sum16(y0, src2d, dst2d)
    y1 = _mid(segp0, y0, b1_0.reshape(1, HID), W2_0, b2_0.reshape(1, HID),
              W1_1)
    segp1 = _seg_sum16(y1, src2d, dst2d)
    out = _head(segp1, y1, b1_1.reshape(1, HID), W2_1, b2_1.reshape(1, HID),
                batch_r, W_out, b_out.reshape(1, 1))
    return jnp.reshape(out, (-1,))


# speedup vs baseline: 19.8890x; 1.5434x over previous
import functools

import jax
import jax.numpy as jnp
from jax import lax
from jax.experimental import pallas as pl
from jax.experimental.pallas import tpu as pltpu
from jax.experimental.pallas import tpu_sc as plsc

N_NODES = 10000
N_EDGES = 320000
D_FEAT = 128
HID = 16
NG = 64

NODES_P = 10240
EDGES_P = 327680
NCORE = 2
NSUB = 16
NW = NCORE * NSUB
CHUNK = 128
NCHUNK = EDGES_P // (NW * CHUNK)
RPT = NODES_P // NSUB
KBUF = 4
RB = 1280
GRID = NODES_P // RB


def _seg_sum16(y_hbm, src2d, dst2d):
    mesh = plsc.VectorSubcoreMesh(core_axis_name="c", subcore_axis_name="s")

    @functools.partial(
        pl.kernel,
        out_type=jax.ShapeDtypeStruct((NCORE, NODES_P, HID), jnp.float32),
        mesh=mesh,
        compiler_params=pltpu.CompilerParams(use_tc_tiling_on_sc=False),
        scratch_types=[
            pltpu.VMEM((NCHUNK, CHUNK), jnp.int32),
            pltpu.VMEM((NCHUNK, CHUNK), jnp.int32),
            pltpu.VMEM((2, KBUF, CHUNK, HID), jnp.float32),
            pltpu.VMEM((RPT, HID), jnp.float32),
            pltpu.VMEM_SHARED((NODES_P, HID), jnp.float32),
            pltpu.SemaphoreType.DMA,
            pltpu.SemaphoreType.DMA,
            pltpu.SemaphoreType.DMA,
        ],
    )
    def k(y_ref, src_ref, dst_ref, out_ref, src_v, dst_v, rows_v, stage_v,
          acc_sh, gsem, ssem0, ssem1):
        cid = lax.axis_index("c")
        sid = lax.axis_index("s")
        wid = cid * NSUB + sid

        def zrow(i, carry):
            stage_v[i, :] = jnp.zeros((HID,), jnp.float32)
            return carry
        lax.fori_loop(0, RPT, zrow, 0)
        pltpu.sync_copy(stage_v, acc_sh.at[pl.ds(sid * RPT, RPT)])

        pltpu.sync_copy(src_ref.at[pl.ds(wid * NCHUNK, NCHUNK)], src_v)
        pltpu.sync_copy(dst_ref.at[pl.ds(wid * NCHUNK, NCHUNK)], dst_v)

        plsc.subcore_barrier()

        def fire_gathers(g, h):
            for b in range(KBUF):
                pltpu.async_copy(
                    y_ref.at[src_v.at[g * KBUF + b]], rows_v.at[h, b], gsem)

        def wait_gathers(g, h):
            for b in range(KBUF):
                pltpu.make_async_copy(
                    y_ref.at[src_v.at[g * KBUF + b]], rows_v.at[h, b],
                    gsem).wait()

        def fire_scatters(g, h):
            sem = ssem0 if h == 0 else ssem1
            for b in range(KBUF):
                pltpu.async_copy(
                    rows_v.at[h, b], acc_sh.at[dst_v.at[g * KBUF + b]],
                    sem, add=True)

        def wait_scatters(g, h):
            sem = ssem0 if h == 0 else ssem1
            for b in range(KBUF):
                pltpu.make_async_copy(
                    rows_v.at[h, b], acc_sh.at[dst_v.at[g * KBUF + b]],
                    sem).wait()

        npairs = NCHUNK // (2 * KBUF)
        fire_gathers(0, 0)

        def pair(t, carry):
            ga = 2 * t
            gb = 2 * t + 1
            wait_gathers(ga, 0)
            fire_scatters(ga, 0)

            @pl.when(t >= 1)
            def _():
                wait_scatters(gb - 2, 1)
            fire_gathers(gb, 1)
            wait_gathers(gb, 1)
            fire_scatters(gb, 1)
            wait_scatters(ga, 0)

            @pl.when(t + 1 < npairs)
            def _():
                fire_gathers(ga + 2, 0)
            return carry
        lax.fori_loop(0, npairs, pair, 0)
        wait_scatters(NCHUNK // KBUF - 1, 1)

        plsc.subcore_barrier()

        pltpu.sync_copy(acc_sh.at[pl.ds(sid * RPT, RPT)], stage_v)
        pltpu.sync_copy(stage_v, out_ref.at[cid, pl.ds(sid * RPT, RPT)])

    return k(y_hbm, src2d, dst2d)


def _proj0(x_pad, w1):
    def body(x_ref, w_ref, o_ref):
        o_ref[...] = jnp.dot(x_ref[...], w_ref[...],
                             preferred_element_type=jnp.float32)

    return pl.pallas_call(
        body,
        grid=(GRID,),
        in_specs=[
            pl.BlockSpec((RB, D_FEAT), lambda i: (i, 0)),
            pl.BlockSpec((D_FEAT, HID), lambda i: (0, 0)),
        ],
        out_specs=pl.BlockSpec((RB, HID), lambda i: (i, 0)),
        out_shape=jax.ShapeDtypeStruct((NODES_P, HID), jnp.float32),
    )(x_pad, w1)


def _mid(segp0, y0, b1_0, w2_0, b2_0, w1_1):
    def body(sp_ref, y_ref, b1_ref, w2_ref, b2_ref, w11_ref, o_ref):
        seg = sp_ref[0] + sp_ref[1]
        t = jnp.maximum(seg + y_ref[...] + b1_ref[...], 0.0)
        t = jnp.dot(t, w2_ref[...], preferred_element_type=jnp.float32)
        t = jnp.maximum(t + b2_ref[...], 0.0)
        o_ref[...] = jnp.dot(t, w11_ref[...],
                             preferred_element_type=jnp.float32)

    return pl.pallas_call(
        body,
        grid=(GRID,),
        in_specs=[
            pl.BlockSpec((NCORE, RB, HID), lambda i: (0, i, 0)),
            pl.BlockSpec((RB, HID), lambda i: (i, 0)),
            pl.BlockSpec((1, HID), lambda i: (0, 0)),
            pl.BlockSpec((HID, HID), lambda i: (0, 0)),
            pl.BlockSpec((1, HID), lambda i: (0, 0)),
            pl.BlockSpec((HID, HID), lambda i: (0, 0)),
        ],
        out_specs=pl.BlockSpec((RB, HID), lambda i: (i, 0)),
        out_shape=jax.ShapeDtypeStruct((NODES_P, HID), jnp.float32),
    )(segp0, y0, b1_0, w2_0, b2_0, w1_1)


def _head(segp1, y1, b1_1, w2_1, b2_1, batch_r, w_out, b_out):
    def body(sp_ref, y_ref, b1_ref, w2_ref, b2_ref, bt_ref, wo_ref, bo_ref,
             o_ref, sums, cnts):
        i = pl.program_id(0)

        @pl.when(i == 0)
        def _init():
            sums[...] = jnp.zeros_like(sums)
            cnts[...] = jnp.zeros_like(cnts)

        seg = sp_ref[0] + sp_ref[1]
        t = jnp.maximum(seg + y_ref[...] + b1_ref[...], 0.0)
        h2 = jnp.dot(t, w2_ref[...], preferred_element_type=jnp.float32)
        h2 = h2 + b2_ref[...]

        bvals = bt_ref[0]
        gids = lax.broadcasted_iota(jnp.int32, (NG, RB), 0)
        onehot = (bvals == gids).astype(jnp.float32)
        sums[...] += jnp.dot(onehot, h2, preferred_element_type=jnp.float32)
        cnts[...] += jnp.dot(onehot, jnp.ones((RB, HID), jnp.float32),
                             preferred_element_type=jnp.float32)

        @pl.when(i == GRID - 1)
        def _final():
            pooled = sums[...] / jnp.maximum(cnts[...], 1.0)
            g = jnp.maximum(pooled, 0.0)
            o_ref[...] = jnp.dot(g, wo_ref[...],
                                 preferred_element_type=jnp.float32)
            o_ref[...] += bo_ref[...]

    return pl.pallas_call(
        body,
        grid=(GRID,),
        in_specs=[
            pl.BlockSpec((NCORE, RB, HID), lambda i: (0, i, 0)),
            pl.BlockSpec((RB, HID), lambda i: (i, 0)),
            pl.BlockSpec((1, HID), lambda i: (0, 0)),
            pl.BlockSpec((HID, HID), lambda i: (0, 0)),
            pl.BlockSpec((1, HID), lambda i: (0, 0)),
            pl.BlockSpec((1, 1, RB), lambda i: (i, 0, 0)),
            pl.BlockSpec((HID, 1), lambda i: (0, 0)),
            pl.BlockSpec((1, 1), lambda i: (0, 0)),
        ],
        out_specs=pl.BlockSpec((NG, 1), lambda i: (0, 0)),
        out_shape=jax.ShapeDtypeStruct((NG, 1), jnp.float32),
        scratch_shapes=[
            pltpu.VMEM((NG, HID), jnp.float32),
            pltpu.VMEM((NG, HID), jnp.float32),
        ],
    )(segp1, y1, b1_1, w2_1, b2_1, batch_r, w_out, b_out)


def kernel(x, edge_index, batch, W1_0, b1_0, W2_0, b2_0, W1_1, b1_1, W2_1,
           b2_1, W_out, b_out):
    x_pad = jnp.pad(x, ((0, NODES_P - N_NODES), (0, 0)))
    src = edge_index[0]
    dst = edge_index[1]
    pad_e = EDGES_P - N_EDGES
    pad_iota = jnp.arange(pad_e, dtype=jnp.int32)
    src2d = jnp.concatenate(
        [src, pad_iota % N_NODES]).reshape(EDGES_P // CHUNK, CHUNK)
    dst2d = jnp.concatenate(
        [dst, N_NODES + pad_iota % (NODES_P - N_NODES)]).reshape(
            EDGES_P // CHUNK, CHUNK)
    batch_r = jnp.concatenate(
        [batch, jnp.full((NODES_P - N_NODES,), NG, jnp.int32)]).reshape(
            GRID, 1, RB)

    y0 = _proj0(x_pad, W1_0)
    segp0 = _seg_sum16(y0, src2d, dst2d)
    y1 = _mid(segp0, y0, b1_0.reshape(1, HID), W2_0, b2_0.reshape(1, HID),
              W1_1)
    segp1 = _seg_sum16(y1, src2d, dst2d)
    out = _head(segp1, y1, b1_1.reshape(1, HID), W2_1, b2_1.reshape(1, HID),
                batch_r, W_out, b_out.reshape(1, 1))
    return jnp.reshape(out, (-1,))
